# baseline (speedup 1.0000x reference)
import functools

import jax
import jax.numpy as jnp
from jax import lax
from jax.experimental import pallas as pl
from jax.experimental.pallas import tpu as pltpu
from jax.experimental.pallas import tpu_sc as plsc

_B = 1024
_C = 100000
_NW = 32
_RPW = _B // _NW
_EDGE_T = (_C // 128)
_EDGE_C = _EDGE_T * 128
_ZW = 12800
_ZTAIL = _EDGE_C - 7 * _ZW

_mesh = plsc.VectorSubcoreMesh(
    core_axis_name="c", subcore_axis_name="s", num_cores=2, num_subcores=16
)


@functools.partial(
    pl.kernel,
    out_type=(
        jax.ShapeDtypeStruct((_B, _C), jnp.float32),
        jax.ShapeDtypeStruct((_B, 3, 224, 224), jnp.float32),
    ),
    mesh=_mesh,
    compiler_params=pltpu.CompilerParams(needs_layout_passes=False),
    scratch_types=[
        pltpu.VMEM((8, _ZW), jnp.float32),
        pltpu.VMEM((_RPW,), jnp.int32),
        pltpu.VMEM((9, 128), jnp.float32),
        pltpu.SemaphoreType.DMA,
        pltpu.SemaphoreType.DMA,
        pltpu.SemaphoreType.DMA,
    ],
)
def _sc_onehot(z_hbm, y_hbm, x_hbm, out_hbm, outx_hbm, ztmpl, yv, pay, fill_sem, fix_sem, x_sem):
    cid = lax.axis_index("c")
    sid = lax.axis_index("s")
    wid = sid * 2 + cid
    row0 = wid * _RPW

    xcopy = pltpu.async_copy(
        x_hbm.at[pl.ds(row0, _RPW)], outx_hbm.at[pl.ds(row0, _RPW)], x_sem
    )
    pltpu.sync_copy(y_hbm.at[pl.ds(row0, _RPW)], yv)
    pltpu.sync_copy(z_hbm, ztmpl)

    copies = []
    for t in range(_RPW // 8):
        r8 = row0 + t * 8
        for cch in range(7):
            copies.append(
                pltpu.async_copy(
                    ztmpl,
                    out_hbm.at[pl.ds(r8, 8), pl.ds(cch * _ZW, _ZW)],
                    fill_sem,
                )
            )
        copies.append(
            pltpu.async_copy(
                ztmpl.at[:, pl.ds(0, _ZTAIL)],
                out_hbm.at[pl.ds(r8, 8), pl.ds(7 * _ZW, _ZTAIL)],
                fill_sem,
            )
        )
    for c in copies:
        c.wait()

    lanes = lax.iota(jnp.int32, 16)
    zf = jnp.zeros((16,), jnp.float32)
    for q in range(8):
        for k in range(8):
            pay[q, pl.ds(k * 16, 16)] = zf

    for s in range(4):
        yvh = yv[pl.ds((s // 2) * 16, 16)]
        rl = (s % 2) * 8
        tv = lax.shift_right_logical(yvh, 7)
        ov = jnp.bitwise_and(yvh, 127)
        inr = jnp.logical_and(lanes >= rl, lanes < rl + 8)
        idx_r = jnp.where(inr, lanes - rl, 8)
        for t in range(8):
            t_t = jnp.sum(jnp.where(lanes == rl + t, tv, 0))

            @pl.when(t_t < _EDGE_T)
            def _(s=s, t_t=t_t, tv=tv, ov=ov, inr=inr, idx_r=idx_r):
                vals = jnp.where(tv == t_t, 1.0, 0.0).astype(jnp.float32)
                plsc.store_scatter(pay, [idx_r, ov], vals, mask=inr)
                pltpu.async_copy(
                    pay.at[pl.ds(0, 8)],
                    out_hbm.at[
                        pl.ds(row0 + s * 8, 8), pl.ds(t_t * 128, 128)
                    ],
                    fix_sem,
                ).wait()
                plsc.store_scatter(pay, [idx_r, ov], zf, mask=inr)

    xcopy.wait()


def _edge_body(oh_ref, y_ref, out_ref):
    col = jax.lax.broadcasted_iota(jnp.int32, out_ref.shape, 1) + _EDGE_C
    out_ref[...] = (col == y_ref[...]).astype(jnp.float32)


def _edge_fix(oh, y2):
    return pl.pallas_call(
        _edge_body,
        grid=(_B // 8,),
        in_specs=[
            pl.BlockSpec(memory_space=pl.ANY),
            pl.BlockSpec((8, 1), lambda i: (i, 0)),
        ],
        out_specs=pl.BlockSpec((8, 128), lambda i: (i, _EDGE_T)),
        out_shape=jax.ShapeDtypeStruct((_B, _C), jnp.float32),
        input_output_aliases={0: 0},
    )(oh, y2)


def kernel(x, y):
    y32 = y.astype(jnp.int32)
    z = jnp.zeros((8, _ZW), jnp.float32)
    oh, x_out = _sc_onehot(z, y32, x)
    oh = _edge_fix(oh, y32.reshape(_B, 1))
    return (x_out, oh)


# speedup vs baseline: 24.2115x; 24.2115x over previous
import functools

import jax
import jax.numpy as jnp
from jax import lax
from jax.experimental import pallas as pl
from jax.experimental.pallas import tpu as pltpu
from jax.experimental.pallas import tpu_sc as plsc

_B = 1024
_C = 100000
_NW = 32
_RPW = _B // _NW
_EDGE_T = (_C // 128)
_EDGE_C = _EDGE_T * 128
_ZW = 12800
_ZTAIL = _EDGE_C - 7 * _ZW

_mesh = plsc.VectorSubcoreMesh(
    core_axis_name="c", subcore_axis_name="s", num_cores=2, num_subcores=16
)


@functools.partial(
    pl.kernel,
    out_type=jax.ShapeDtypeStruct((_B, _C), jnp.float32),
    mesh=_mesh,
    compiler_params=pltpu.CompilerParams(needs_layout_passes=False),
    scratch_types=[
        pltpu.VMEM((8, _ZW), jnp.float32),
        pltpu.VMEM((_RPW,), jnp.int32),
        pltpu.VMEM((9, 128), jnp.float32),
        pltpu.SemaphoreType.DMA,
        pltpu.SemaphoreType.DMA,
    ],
)
def _sc_onehot(z_hbm, y_hbm, out_hbm, ztmpl, yv, pay, fill_sem, fix_sem):
    cid = lax.axis_index("c")
    sid = lax.axis_index("s")
    wid = sid * 2 + cid
    row0 = wid * _RPW

    pltpu.sync_copy(y_hbm.at[pl.ds(row0, _RPW)], yv)
    pltpu.sync_copy(z_hbm, ztmpl)

    copies = []
    for t in range(_RPW // 8):
        r8 = row0 + t * 8
        for cch in range(7):
            copies.append(
                pltpu.async_copy(
                    ztmpl,
                    out_hbm.at[pl.ds(r8, 8), pl.ds(cch * _ZW, _ZW)],
                    fill_sem,
                )
            )
        copies.append(
            pltpu.async_copy(
                ztmpl.at[:, pl.ds(0, _ZTAIL)],
                out_hbm.at[pl.ds(r8, 8), pl.ds(7 * _ZW, _ZTAIL)],
                fill_sem,
            )
        )
    for c in copies:
        c.wait()

    lanes = lax.iota(jnp.int32, 16)
    zf = jnp.zeros((16,), jnp.float32)
    for q in range(8):
        for k in range(8):
            pay[q, pl.ds(k * 16, 16)] = zf

    for s in range(4):
        yvh = yv[pl.ds((s // 2) * 16, 16)]
        rl = (s % 2) * 8
        tv = lax.shift_right_logical(yvh, 7)
        ov = jnp.bitwise_and(yvh, 127)
        inr = jnp.logical_and(lanes >= rl, lanes < rl + 8)
        idx_r = jnp.where(inr, lanes - rl, 8)
        for t in range(8):
            t_t = jnp.sum(jnp.where(lanes == rl + t, tv, 0))

            @pl.when(t_t < _EDGE_T)
            def _(s=s, t_t=t_t, tv=tv, ov=ov, inr=inr, idx_r=idx_r):
                vals = jnp.where(tv == t_t, 1.0, 0.0).astype(jnp.float32)
                plsc.store_scatter(pay, [idx_r, ov], vals, mask=inr)
                pltpu.async_copy(
                    pay.at[pl.ds(0, 8)],
                    out_hbm.at[
                        pl.ds(row0 + s * 8, 8), pl.ds(t_t * 128, 128)
                    ],
                    fix_sem,
                ).wait()
                plsc.store_scatter(pay, [idx_r, ov], zf, mask=inr)


def _edge_body(oh_ref, y_ref, out_ref):
    col = jax.lax.broadcasted_iota(jnp.int32, out_ref.shape, 1) + _EDGE_C
    out_ref[...] = (col == y_ref[...]).astype(jnp.float32)


def _edge_fix(oh, y2):
    return pl.pallas_call(
        _edge_body,
        grid=(_B // 8,),
        in_specs=[
            pl.BlockSpec(memory_space=pl.ANY),
            pl.BlockSpec((8, 1), lambda i: (i, 0)),
        ],
        out_specs=pl.BlockSpec((8, 128), lambda i: (i, _EDGE_T)),
        out_shape=jax.ShapeDtypeStruct((_B, _C), jnp.float32),
        input_output_aliases={0: 0},
    )(oh, y2)


def kernel(x, y):
    y32 = y.astype(jnp.int32)
    z = jnp.zeros((8, _ZW), jnp.float32)
    oh = _sc_onehot(z, y32)
    oh = _edge_fix(oh, y32.reshape(_B, 1))
    return (x, oh)
